# baseline (speedup 1.0000x reference)
import functools

import jax
import jax.numpy as jnp
from jax import lax
from jax.experimental import pallas as pl
from jax.experimental.pallas import tpu as pltpu
from jax.experimental.pallas import tpu_sc as plsc

B = 64
N = 2048
D = 3
S = 512
L = 16
CHUNKS = N // L
UNROLL = 8
NUM_CORES = 2
NUM_SUBCORES = 16
NW = NUM_CORES * NUM_SUBCORES
PER_W = B // NW


def _fps_one_cloud(xv, dist, samp, idxv):
  inf_v = jnp.full((L,), jnp.inf, dtype=jnp.float32)
  lanes = lax.broadcasted_iota(jnp.int32, (L,), 0)
  lane0 = lanes == 0
  zero_i = jnp.zeros((L,), dtype=jnp.int32)
  neg_inf_v = jnp.full((L,), -jnp.inf, dtype=jnp.float32)
  int_max_v = jnp.full((L,), jnp.int32(2147483647), dtype=jnp.int32)

  def init_body(c, carry):
    dist[pl.ds(c * L, L)] = inf_v
    return carry

  lax.fori_loop(0, CHUNKS, init_body, 0)

  def pick(j_vec, i_vec):
    plsc.store_scatter(idxv, [i_vec], j_vec, mask=lane0)
    qx = plsc.load_gather(xv, [j_vec])
    qy = plsc.load_gather(xv, [j_vec + N])
    qz = plsc.load_gather(xv, [j_vec + 2 * N])
    return qx, qy, qz

  px = zero_i.astype(jnp.float32) + xv[pl.ds(0, L)][0]
  py = zero_i.astype(jnp.float32) + xv[pl.ds(N, L)][0]
  pz = zero_i.astype(jnp.float32) + xv[pl.ds(2 * N, L)][0]
  plsc.store_scatter(idxv, [zero_i], zero_i, mask=lane0)

  def iter_body(i, carry):
    px, py, pz = carry

    @plsc.parallel_loop(0, CHUNKS, 1, unroll=UNROLL, carry=neg_inf_v)
    def chunk_body(c, bv):
      off = c * L
      dx = xv[pl.ds(off, L)] - px
      dy = xv[pl.ds(off + N, L)] - py
      dz = xv[pl.ds(off + 2 * N, L)] - pz
      d = dx * dx + (dy * dy + dz * dz)
      dm = jnp.minimum(dist[pl.ds(off, L)], d)
      dist[pl.ds(off, L)] = dm
      return jnp.maximum(bv, dm)

    m = jnp.max(chunk_body)

    @plsc.parallel_loop(0, CHUNKS, 1, unroll=UNROLL, carry=int_max_v)
    def find_body(c, bi):
      off = c * L
      dmv = dist[pl.ds(off, L)]
      return jnp.minimum(bi, jnp.where(dmv == m, off + lanes, int_max_v))

    j_vec = zero_i + jnp.min(find_body)
    return pick(j_vec, zero_i + i)

  lax.fori_loop(1, S, iter_body, (px, py, pz))

  @plsc.parallel_loop(0, S // L, 1, unroll=4)
  def samp_body(c):
    base = c * L
    iv = idxv[pl.ds(base, L)]
    samp[pl.ds(base, L)] = plsc.load_gather(xv, [iv])
    samp[pl.ds(base + S, L)] = plsc.load_gather(xv, [iv + N])
    samp[pl.ds(base + 2 * S, L)] = plsc.load_gather(xv, [iv + 2 * N])
  del samp_body


@functools.partial(
    pl.kernel,
    mesh=plsc.VectorSubcoreMesh(core_axis_name="c", subcore_axis_name="s"),
    compiler_params=pltpu.CompilerParams(needs_layout_passes=False),
    out_type=[
        jax.ShapeDtypeStruct((B, D * S), jnp.float32),
        jax.ShapeDtypeStruct((B, S), jnp.int32),
    ],
    scratch_types=[
        pltpu.VMEM((D * N,), jnp.float32),
        pltpu.VMEM((N,), jnp.float32),
        pltpu.VMEM((D * S,), jnp.float32),
        pltpu.VMEM((S,), jnp.int32),
    ],
)
def _fps_sc(x_hbm, samp_hbm, idx_hbm, xv, dist, samp, idxv):
  wid = lax.axis_index("s") * NUM_CORES + lax.axis_index("c")
  for k in range(PER_W):
    b = wid * PER_W + k
    pltpu.sync_copy(x_hbm.at[b], xv)
    _fps_one_cloud(xv, dist, samp, idxv)
    pltpu.sync_copy(samp, samp_hbm.at[b])
    pltpu.sync_copy(idxv, idx_hbm.at[b])


@jax.jit
def kernel(x):
  xt = jnp.swapaxes(x, 1, 2).reshape(B, D * N)
  samp_t, idx = _fps_sc(xt)
  sampled = jnp.swapaxes(samp_t.reshape(B, D, S), 1, 2)
  return sampled, idx


# speedup vs baseline: 1.1799x; 1.1799x over previous
import functools

import jax
import jax.numpy as jnp
from jax import lax
from jax.experimental import pallas as pl
from jax.experimental.pallas import tpu as pltpu
from jax.experimental.pallas import tpu_sc as plsc

B = 64
N = 2048
D = 3
S = 512
L = 16
CHUNKS = N // L
UNROLL = 8
NUM_CORES = 2
NUM_SUBCORES = 16
NW = NUM_CORES * NUM_SUBCORES
PER_W = B // NW


def _fps_one_cloud(xv, dist, samp, idxv):
  inf_v = jnp.full((L,), jnp.inf, dtype=jnp.float32)
  lanes = lax.broadcasted_iota(jnp.int32, (L,), 0)
  lane0 = lanes == 0
  zero_i = jnp.zeros((L,), dtype=jnp.int32)
  neg_inf_v = jnp.full((L,), -jnp.inf, dtype=jnp.float32)
  int_max_v = jnp.full((L,), jnp.int32(2147483647), dtype=jnp.int32)

  def init_body(c, carry):
    dist[pl.ds(c * L, L)] = inf_v
    return carry

  lax.fori_loop(0, CHUNKS, init_body, 0)

  def pick(j_vec, i_vec):
    plsc.store_scatter(idxv, [i_vec], j_vec, mask=lane0)
    qx = plsc.load_gather(xv, [j_vec])
    qy = plsc.load_gather(xv, [j_vec + N])
    qz = plsc.load_gather(xv, [j_vec + 2 * N])
    return qx, qy, qz

  px = zero_i.astype(jnp.float32) + xv[pl.ds(0, L)][0]
  py = zero_i.astype(jnp.float32) + xv[pl.ds(N, L)][0]
  pz = zero_i.astype(jnp.float32) + xv[pl.ds(2 * N, L)][0]
  plsc.store_scatter(idxv, [zero_i], zero_i, mask=lane0)

  def iter_body(i, carry):
    px, py, pz = carry

    def upd(off, bv, bi):
      dx = xv[pl.ds(off, L)] - px
      dy = xv[pl.ds(off + N, L)] - py
      dz = xv[pl.ds(off + 2 * N, L)] - pz
      d = dx * dx + (dy * dy + dz * dz)
      dm = jnp.minimum(dist[pl.ds(off, L)], d)
      dist[pl.ds(off, L)] = dm
      idx = off + lanes
      better = (dm > bv) | ((dm == bv) & (idx < bi))
      return jnp.where(better, dm, bv), jnp.where(better, idx, bi)

    @plsc.parallel_loop(0, CHUNKS, 2, unroll=UNROLL // 2,
                        carry=(neg_inf_v, zero_i, neg_inf_v, zero_i))
    def chunk_body(c, c_carry):
      bv0, bi0, bv1, bi1 = c_carry
      bv0, bi0 = upd(c * L, bv0, bi0)
      bv1, bi1 = upd(c * L + L, bv1, bi1)
      return (bv0, bi0, bv1, bi1)

    bv0, bi0, bv1, bi1 = chunk_body
    better = (bv1 > bv0) | ((bv1 == bv0) & (bi1 < bi0))
    bv = jnp.where(better, bv1, bv0)
    bi = jnp.where(better, bi1, bi0)
    m = jnp.max(bv)
    cand = jnp.where(bv == m, bi, int_max_v)
    j_vec = zero_i + jnp.min(cand)
    return pick(j_vec, zero_i + i)

  lax.fori_loop(1, S, iter_body, (px, py, pz))

  @plsc.parallel_loop(0, S // L, 1, unroll=4)
  def samp_body(c):
    base = c * L
    iv = idxv[pl.ds(base, L)]
    samp[pl.ds(base, L)] = plsc.load_gather(xv, [iv])
    samp[pl.ds(base + S, L)] = plsc.load_gather(xv, [iv + N])
    samp[pl.ds(base + 2 * S, L)] = plsc.load_gather(xv, [iv + 2 * N])
  del samp_body


@functools.partial(
    pl.kernel,
    mesh=plsc.VectorSubcoreMesh(core_axis_name="c", subcore_axis_name="s"),
    compiler_params=pltpu.CompilerParams(needs_layout_passes=False),
    out_type=[
        jax.ShapeDtypeStruct((B, D * S), jnp.float32),
        jax.ShapeDtypeStruct((B, S), jnp.int32),
    ],
    scratch_types=[
        pltpu.VMEM((D * N,), jnp.float32),
        pltpu.VMEM((N,), jnp.float32),
        pltpu.VMEM((D * S,), jnp.float32),
        pltpu.VMEM((S,), jnp.int32),
    ],
)
def _fps_sc(x_hbm, samp_hbm, idx_hbm, xv, dist, samp, idxv):
  wid = lax.axis_index("s") * NUM_CORES + lax.axis_index("c")
  for k in range(PER_W):
    b = wid * PER_W + k
    pltpu.sync_copy(x_hbm.at[b], xv)
    _fps_one_cloud(xv, dist, samp, idxv)
    pltpu.sync_copy(samp, samp_hbm.at[b])
    pltpu.sync_copy(idxv, idx_hbm.at[b])


@jax.jit
def kernel(x):
  xt = jnp.swapaxes(x, 1, 2).reshape(B, D * N)
  samp_t, idx = _fps_sc(xt)
  sampled = jnp.swapaxes(samp_t.reshape(B, D, S), 1, 2)
  return sampled, idx
